# Initial kernel scaffold; baseline (speedup 1.0000x reference)
import jax
import jax.numpy as jnp
from jax.experimental import pallas as pl


def kernel(x, edge_index, batch, W1, b1, W2, b2, W3, b3, Wl, bl):
    raise NotImplementedError("write your pallas kernel here")


# speedup vs baseline: 12.2592x; 12.2592x over previous
import jax
import jax.numpy as jnp
from jax import lax
from jax.experimental import pallas as pl
from jax.experimental.pallas import tpu as pltpu
from jax.experimental.pallas import tpu_sc as plsc

NC, NS, LANES = 2, 16, 16
NV = 50000
NE = 800000
FIN = 8
HID = 64
NQ = 4
HQ = HID // NQ
NPASS = NQ // NC
NGRAPH = 256

NP = 50176
NA = 51200
CHUNK = 128
CHUNKS = 392
EP = NS * CHUNKS * CHUNK
NBUF = 4
OUTER = CHUNKS // NBUF
DROWS = CHUNKS // NC

TN = 512
GRID_N = NP // TN

_f32 = jnp.float32
_mesh = plsc.VectorSubcoreMesh(core_axis_name="c", subcore_axis_name="s",
                               num_cores=NC, num_subcores=NS)
_sc_params = pltpu.CompilerParams(needs_layout_passes=False,
                                  use_tc_tiling_on_sc=False)


def _deg_body(dst_hbm, deg_out, dslab, hist):
    ci = lax.axis_index("c")
    si = lax.axis_index("s")
    wid = si * NC + ci
    pltpu.sync_copy(dst_hbm.at[wid], dslab)
    zeros = jnp.zeros((LANES,), _f32)
    ones = jnp.ones((LANES,), _f32)

    def _zero(i, carry):
        hist[pl.ds(i * LANES, LANES)] = zeros
        return carry
    lax.fori_loop(0, NA // LANES, _zero, 0)

    def _row(r, carry):
        def _grp(q, carry2):
            idx = dslab[r, pl.ds(q * LANES, LANES)]
            plsc.addupdate_scatter(hist, [idx], ones)
            return carry2
        return lax.fori_loop(0, CHUNK // LANES, _grp, carry)
    lax.fori_loop(0, DROWS, _row, 0)

    pltpu.sync_copy(hist, deg_out.at[wid])


_deg_call = pl.kernel(
    _deg_body,
    out_type=pltpu.HBM((NC * NS, NA), _f32),
    mesh=_mesh,
    compiler_params=_sc_params,
    scratch_types=[
        pltpu.VMEM((DROWS, CHUNK), jnp.int32),
        pltpu.VMEM((NA,), _f32),
    ],
)


def _agg_body(y_hbm, src_hbm, dst_hbm, z_hbm,
              sb0, sb1, sb2, sb3, db0, db1, db2, db3,
              g0, g1, g2, g3, zbuf, acc,
              is0, is1, is2, is3, gs0, gs1, gs2, gs3, ss0, ss1, ss2, ss3):
    ci = lax.axis_index("c")
    si = lax.axis_index("s")
    sb = [sb0, sb1, sb2, sb3]
    db = [db0, db1, db2, db3]
    gb = [g0, g1, g2, g3]
    isem = [is0, is1, is2, is3]
    gsem = [gs0, gs1, gs2, gs3]
    ssem = [ss0, ss1, ss2, ss3]

    zeros = jnp.zeros((LANES,), _f32)

    def _zb(k, carry):
        zbuf[k, pl.ds(0, LANES)] = zeros
        return carry
    lax.fori_loop(0, CHUNK, _zb, 0)

    arows = NA // NS
    zrows = NP // NS
    base = si * arows

    for p in range(NPASS):
        qi = p * NC + ci

        def _zs(t, carry):
            pltpu.sync_copy(zbuf, acc.at[pl.ds(base + t * CHUNK, CHUNK)])
            return carry
        lax.fori_loop(0, arows // CHUNK, _zs, 0)
        plsc.subcore_barrier()

        yq = y_hbm.at[qi]
        for b in range(NBUF):
            pltpu.async_copy(src_hbm.at[si, b], sb[b], isem[b])
            pltpu.async_copy(dst_hbm.at[si, b], db[b], isem[b])

        def _outer(o, carry):
            for b in range(NBUF):
                c = o * NBUF + b
                pltpu.make_async_copy(src_hbm.at[si, c], sb[b],
                                      isem[b]).wait()
                pltpu.make_async_copy(dst_hbm.at[si, c], db[b],
                                      isem[b]).wait()
                pltpu.async_copy(yq.at[sb[b].at[0]], gb[b], gsem[b])
            for b in range(NBUF):
                pltpu.make_async_copy(yq.at[sb[b].at[0]], gb[b],
                                      gsem[b]).wait()
                pltpu.async_copy(gb[b], acc.at[db[b].at[0]], ssem[b],
                                 add=True)
            for b in range(NBUF):
                c = o * NBUF + b
                pltpu.make_async_copy(gb[b], acc.at[db[b].at[0]],
                                      ssem[b]).wait()

                @pl.when(c + NBUF < CHUNKS)
                def _():
                    pltpu.async_copy(src_hbm.at[si, c + NBUF], sb[b],
                                     isem[b])
                    pltpu.async_copy(dst_hbm.at[si, c + NBUF], db[b],
                                     isem[b])
            return carry
        lax.fori_loop(0, OUTER, _outer, 0)
        plsc.subcore_barrier()

        pltpu.sync_copy(acc.at[pl.ds(si * zrows, zrows)],
                        z_hbm.at[qi, pl.ds(si * zrows, zrows)])
        plsc.subcore_barrier()


_agg_call = pl.kernel(
    _agg_body,
    out_type=pltpu.HBM((NQ, NP, HQ), _f32),
    mesh=_mesh,
    compiler_params=_sc_params,
    scratch_types=(
        [pltpu.VMEM((1, CHUNK), jnp.int32)] * 8
        + [pltpu.VMEM((CHUNK, HQ), _f32)] * 5
        + [pltpu.VMEM_SHARED((NA, HQ), _f32)]
        + [pltpu.SemaphoreType.DMA] * 12
    ),
)


def _split_q(y):
    return [y[:, q * HQ:(q + 1) * HQ] for q in range(NQ)]


def _t1_body(x_ref, deg_ref, w_ref, y_ref, dis_ref):
    deg = jnp.sum(deg_ref[...], axis=0) + 1.0
    dis = lax.rsqrt(deg)
    y = dis * jnp.dot(x_ref[...], w_ref[...], preferred_element_type=_f32)
    for q, yq in enumerate(_split_q(y)):
        y_ref[q] = yq
    dis_ref[...] = dis


def _t1_call(x_p, degp, W1):
    return pl.pallas_call(
        _t1_body,
        grid=(GRID_N,),
        in_specs=[
            pl.BlockSpec((TN, FIN), lambda i: (i, 0)),
            pl.BlockSpec((NC * NS, TN, 1), lambda i: (0, i, 0)),
            pl.BlockSpec((FIN, HID), lambda i: (0, 0)),
        ],
        out_specs=[
            pl.BlockSpec((NQ, TN, HQ), lambda i: (0, i, 0)),
            pl.BlockSpec((TN, 1), lambda i: (i, 0)),
        ],
        out_shape=[
            jax.ShapeDtypeStruct((NQ, NP, HQ), _f32),
            jax.ShapeDtypeStruct((NP, 1), _f32),
        ],
    )(x_p, degp, W1)


def _cat_q(z_ref, y_ref):
    return jnp.concatenate([z_ref[q] + y_ref[q] for q in range(NQ)], axis=1)


def _t23_body(z_ref, y_ref, dis_ref, b_ref, w_ref, yn_ref):
    dis = dis_ref[...]
    h = jnp.maximum(dis * _cat_q(z_ref, y_ref) + b_ref[...], 0.0)
    yn = dis * jnp.dot(h, w_ref[...], preferred_element_type=_f32)
    for q, yq in enumerate(_split_q(yn)):
        yn_ref[q] = yq


def _t23_call(z, y, dis, b_prev, W_next):
    return pl.pallas_call(
        _t23_body,
        grid=(GRID_N,),
        in_specs=[
            pl.BlockSpec((NQ, TN, HQ), lambda i: (0, i, 0)),
            pl.BlockSpec((NQ, TN, HQ), lambda i: (0, i, 0)),
            pl.BlockSpec((TN, 1), lambda i: (i, 0)),
            pl.BlockSpec((1, HID), lambda i: (0, 0)),
            pl.BlockSpec((HID, HID), lambda i: (0, 0)),
        ],
        out_specs=pl.BlockSpec((NQ, TN, HQ), lambda i: (0, i, 0)),
        out_shape=jax.ShapeDtypeStruct((NQ, NP, HQ), _f32),
    )(z, y, dis, b_prev, W_next)


def _t4_body(z_ref, y_ref, dis_ref, b_ref, wl_ref, bl_ref, batch_ref,
             out_ref, sums_ref, cnt_ref):
    i = pl.program_id(0)

    @pl.when(i == 0)
    def _():
        sums_ref[...] = jnp.zeros_like(sums_ref)
        cnt_ref[...] = jnp.zeros_like(cnt_ref)

    h3 = dis_ref[...] * _cat_q(z_ref, y_ref) + b_ref[...]
    g = jnp.dot(h3, wl_ref[...], preferred_element_type=_f32)
    bb = batch_ref[0, 0, :]
    oh = (bb[:, None] ==
          lax.broadcasted_iota(jnp.int32, (TN, NGRAPH), 1)).astype(_f32)
    sums_ref[...] += lax.dot_general(oh, g, (((0,), (0,)), ((), ())),
                                     preferred_element_type=_f32)
    cnt_ref[...] += jnp.sum(oh, axis=0)[:, None]

    @pl.when(i == GRID_N - 1)
    def _():
        cnt = cnt_ref[...]
        cnt = jnp.where(cnt > 0, cnt, 1.0)
        out_ref[...] = jax.nn.sigmoid(sums_ref[...] / cnt + bl_ref[...])


def _t4_call(z3, y3, dis, b3, Wl, blr, batch_p):
    return pl.pallas_call(
        _t4_body,
        grid=(GRID_N,),
        in_specs=[
            pl.BlockSpec((NQ, TN, HQ), lambda i: (0, i, 0)),
            pl.BlockSpec((NQ, TN, HQ), lambda i: (0, i, 0)),
            pl.BlockSpec((TN, 1), lambda i: (i, 0)),
            pl.BlockSpec((1, HID), lambda i: (0, 0)),
            pl.BlockSpec((HID, 1), lambda i: (0, 0)),
            pl.BlockSpec((1, 1), lambda i: (0, 0)),
            pl.BlockSpec((1, 1, TN), lambda i: (i, 0, 0)),
        ],
        out_specs=pl.BlockSpec((NGRAPH, 1), lambda i: (0, 0)),
        out_shape=jax.ShapeDtypeStruct((NGRAPH, 1), _f32),
        scratch_shapes=[
            pltpu.VMEM((NGRAPH, 1), _f32),
            pltpu.VMEM((NGRAPH, 1), _f32),
        ],
    )(z3, y3, dis, b3, Wl, blr, batch_p)


def kernel(x, edge_index, batch, W1, b1, W2, b2, W3, b3, Wl, bl):
    src = edge_index[0]
    dst = edge_index[1]
    pad_e = EP - NE
    src_p = jnp.concatenate([src, jnp.zeros((pad_e,), jnp.int32)])
    dst_p = jnp.concatenate(
        [dst, NP + (jnp.arange(pad_e, dtype=jnp.int32) % 128)])
    src_t = src_p.reshape(NS, CHUNKS, 1, CHUNK)
    dst_t = dst_p.reshape(NS, CHUNKS, 1, CHUNK)

    x_p = jnp.pad(x, ((0, NP - NV), (0, 0)))
    batch_p = jnp.pad(batch, (0, NP - NV),
                      constant_values=NGRAPH).reshape(GRID_N, 1, TN)

    deg_parts = _deg_call(dst_p.reshape(NC * NS, DROWS, CHUNK))
    degp = deg_parts[:, :NP, None]

    y1, dis = _t1_call(x_p, degp, W1)
    z1 = _agg_call(y1, src_t, dst_t)
    y2 = _t23_call(z1, y1, dis, b1.reshape(1, HID), W2)
    z2 = _agg_call(y2, src_t, dst_t)
    y3 = _t23_call(z2, y2, dis, b2.reshape(1, HID), W3)
    z3 = _agg_call(y3, src_t, dst_t)
    return _t4_call(z3, y3, dis, b3.reshape(1, HID), Wl,
                    bl.reshape(1, 1), batch_p)


# speedup vs baseline: 13.9864x; 1.1409x over previous
import jax
import jax.numpy as jnp
from jax import lax
from jax.experimental import pallas as pl
from jax.experimental.pallas import tpu as pltpu
from jax.experimental.pallas import tpu_sc as plsc

NC, NS, LANES = 2, 16, 16
NV = 50000
NE = 800000
FIN = 8
HID = 64
NQ = 4
HQ = HID // NQ
NPASS = NQ // NC
NGRAPH = 256

NP = 50176
NA = 51200
CHUNK = 128
CHUNKS = 392
EP = NS * CHUNKS * CHUNK
NBUF = 8
OUTER = CHUNKS // NBUF
DROWS = CHUNKS // NC

TN = 512
GRID_N = NP // TN

_f32 = jnp.float32
_mesh = plsc.VectorSubcoreMesh(core_axis_name="c", subcore_axis_name="s",
                               num_cores=NC, num_subcores=NS)
_sc_params = pltpu.CompilerParams(needs_layout_passes=False,
                                  use_tc_tiling_on_sc=False)


def _deg_body(dst_hbm, deg_out, dslab, hist):
    ci = lax.axis_index("c")
    si = lax.axis_index("s")
    wid = si * NC + ci
    pltpu.sync_copy(dst_hbm.at[wid], dslab)
    zeros = jnp.zeros((LANES,), _f32)
    ones = jnp.ones((LANES,), _f32)

    def _zero(i, carry):
        hist[pl.ds(i * LANES, LANES)] = zeros
        return carry
    lax.fori_loop(0, NA // LANES, _zero, 0)

    def _row(r, carry):
        def _grp(q, carry2):
            idx = dslab[r, pl.ds(q * LANES, LANES)]
            plsc.addupdate_scatter(hist, [idx], ones)
            return carry2
        return lax.fori_loop(0, CHUNK // LANES, _grp, carry)
    lax.fori_loop(0, DROWS, _row, 0)

    pltpu.sync_copy(hist, deg_out.at[wid])


_deg_call = pl.kernel(
    _deg_body,
    out_type=pltpu.HBM((NC * NS, NA), _f32),
    mesh=_mesh,
    compiler_params=_sc_params,
    scratch_types=[
        pltpu.VMEM((DROWS, CHUNK), jnp.int32),
        pltpu.VMEM((NA,), _f32),
    ],
)


def _agg_body(y_hbm, src_hbm, dst_hbm, z_hbm, *scr):
    ci = lax.axis_index("c")
    si = lax.axis_index("s")
    sb = scr[0:NBUF]
    db = scr[NBUF:2 * NBUF]
    gb = scr[2 * NBUF:3 * NBUF]
    zbuf = scr[3 * NBUF]
    acc = scr[3 * NBUF + 1]
    isem = scr[3 * NBUF + 2:4 * NBUF + 2]
    gsem = scr[4 * NBUF + 2:5 * NBUF + 2]
    ssem = scr[5 * NBUF + 2:6 * NBUF + 2]

    zeros = jnp.zeros((LANES,), _f32)

    def _zb(k, carry):
        zbuf[k, pl.ds(0, LANES)] = zeros
        return carry
    lax.fori_loop(0, CHUNK, _zb, 0)

    arows = NA // NS
    zrows = NP // NS
    base = si * arows

    for p in range(NPASS):
        qi = p * NC + ci

        def _zs(t, carry):
            pltpu.sync_copy(zbuf, acc.at[pl.ds(base + t * CHUNK, CHUNK)])
            return carry
        lax.fori_loop(0, arows // CHUNK, _zs, 0)
        plsc.subcore_barrier()

        yq = y_hbm.at[qi]
        for b in range(NBUF):
            pltpu.async_copy(src_hbm.at[si, b], sb[b], isem[b])
            pltpu.async_copy(dst_hbm.at[si, b], db[b], isem[b])

        def _outer(o, carry):
            for b in range(NBUF):
                c = o * NBUF + b
                pltpu.make_async_copy(src_hbm.at[si, c], sb[b],
                                      isem[b]).wait()
                pltpu.make_async_copy(dst_hbm.at[si, c], db[b],
                                      isem[b]).wait()
                pltpu.async_copy(yq.at[sb[b].at[0]], gb[b], gsem[b])
            for b in range(NBUF):
                pltpu.make_async_copy(yq.at[sb[b].at[0]], gb[b],
                                      gsem[b]).wait()
                pltpu.async_copy(gb[b], acc.at[db[b].at[0]], ssem[b],
                                 add=True)
            for b in range(NBUF):
                c = o * NBUF + b
                pltpu.make_async_copy(gb[b], acc.at[db[b].at[0]],
                                      ssem[b]).wait()

                @pl.when(c + NBUF < CHUNKS)
                def _():
                    pltpu.async_copy(src_hbm.at[si, c + NBUF], sb[b],
                                     isem[b])
                    pltpu.async_copy(dst_hbm.at[si, c + NBUF], db[b],
                                     isem[b])
            return carry
        lax.fori_loop(0, OUTER, _outer, 0)
        plsc.subcore_barrier()

        pltpu.sync_copy(acc.at[pl.ds(si * zrows, zrows)],
                        z_hbm.at[qi, pl.ds(si * zrows, zrows)])
        plsc.subcore_barrier()


_agg_call = pl.kernel(
    _agg_body,
    out_type=pltpu.HBM((NQ, NP, HQ), _f32),
    mesh=_mesh,
    compiler_params=_sc_params,
    scratch_types=(
        [pltpu.VMEM((1, CHUNK), jnp.int32)] * (2 * NBUF)
        + [pltpu.VMEM((CHUNK, HQ), _f32)] * (NBUF + 1)
        + [pltpu.VMEM_SHARED((NA, HQ), _f32)]
        + [pltpu.SemaphoreType.DMA] * (3 * NBUF)
    ),
)


def _split_q(y):
    return [y[:, q * HQ:(q + 1) * HQ] for q in range(NQ)]


def _t1_body(x_ref, deg_ref, w_ref, y_ref, dis_ref):
    deg = jnp.sum(deg_ref[...], axis=0) + 1.0
    dis = lax.rsqrt(deg)
    y = dis * jnp.dot(x_ref[...], w_ref[...], preferred_element_type=_f32)
    for q, yq in enumerate(_split_q(y)):
        y_ref[q] = yq
    dis_ref[...] = dis


def _t1_call(x_p, degp, W1):
    return pl.pallas_call(
        _t1_body,
        grid=(GRID_N,),
        in_specs=[
            pl.BlockSpec((TN, FIN), lambda i: (i, 0)),
            pl.BlockSpec((NC * NS, TN, 1), lambda i: (0, i, 0)),
            pl.BlockSpec((FIN, HID), lambda i: (0, 0)),
        ],
        out_specs=[
            pl.BlockSpec((NQ, TN, HQ), lambda i: (0, i, 0)),
            pl.BlockSpec((TN, 1), lambda i: (i, 0)),
        ],
        out_shape=[
            jax.ShapeDtypeStruct((NQ, NP, HQ), _f32),
            jax.ShapeDtypeStruct((NP, 1), _f32),
        ],
    )(x_p, degp, W1)


def _cat_q(z_ref, y_ref):
    return jnp.concatenate([z_ref[q] + y_ref[q] for q in range(NQ)], axis=1)


def _t23_body(z_ref, y_ref, dis_ref, b_ref, w_ref, yn_ref):
    dis = dis_ref[...]
    h = jnp.maximum(dis * _cat_q(z_ref, y_ref) + b_ref[...], 0.0)
    yn = dis * jnp.dot(h, w_ref[...], preferred_element_type=_f32)
    for q, yq in enumerate(_split_q(yn)):
        yn_ref[q] = yq


def _t23_call(z, y, dis, b_prev, W_next):
    return pl.pallas_call(
        _t23_body,
        grid=(GRID_N,),
        in_specs=[
            pl.BlockSpec((NQ, TN, HQ), lambda i: (0, i, 0)),
            pl.BlockSpec((NQ, TN, HQ), lambda i: (0, i, 0)),
            pl.BlockSpec((TN, 1), lambda i: (i, 0)),
            pl.BlockSpec((1, HID), lambda i: (0, 0)),
            pl.BlockSpec((HID, HID), lambda i: (0, 0)),
        ],
        out_specs=pl.BlockSpec((NQ, TN, HQ), lambda i: (0, i, 0)),
        out_shape=jax.ShapeDtypeStruct((NQ, NP, HQ), _f32),
    )(z, y, dis, b_prev, W_next)


def _t4_body(z_ref, y_ref, dis_ref, b_ref, wl_ref, bl_ref, batch_ref,
             out_ref, sums_ref, cnt_ref):
    i = pl.program_id(0)

    @pl.when(i == 0)
    def _():
        sums_ref[...] = jnp.zeros_like(sums_ref)
        cnt_ref[...] = jnp.zeros_like(cnt_ref)

    h3 = dis_ref[...] * _cat_q(z_ref, y_ref) + b_ref[...]
    g = jnp.dot(h3, wl_ref[...], preferred_element_type=_f32)
    bb = batch_ref[0, 0, :]
    oh = (bb[:, None] ==
          lax.broadcasted_iota(jnp.int32, (TN, NGRAPH), 1)).astype(_f32)
    sums_ref[...] += lax.dot_general(oh, g, (((0,), (0,)), ((), ())),
                                     preferred_element_type=_f32)
    cnt_ref[...] += jnp.sum(oh, axis=0)[:, None]

    @pl.when(i == GRID_N - 1)
    def _():
        cnt = cnt_ref[...]
        cnt = jnp.where(cnt > 0, cnt, 1.0)
        out_ref[...] = jax.nn.sigmoid(sums_ref[...] / cnt + bl_ref[...])


def _t4_call(z3, y3, dis, b3, Wl, blr, batch_p):
    return pl.pallas_call(
        _t4_body,
        grid=(GRID_N,),
        in_specs=[
            pl.BlockSpec((NQ, TN, HQ), lambda i: (0, i, 0)),
            pl.BlockSpec((NQ, TN, HQ), lambda i: (0, i, 0)),
            pl.BlockSpec((TN, 1), lambda i: (i, 0)),
            pl.BlockSpec((1, HID), lambda i: (0, 0)),
            pl.BlockSpec((HID, 1), lambda i: (0, 0)),
            pl.BlockSpec((1, 1), lambda i: (0, 0)),
            pl.BlockSpec((1, 1, TN), lambda i: (i, 0, 0)),
        ],
        out_specs=pl.BlockSpec((NGRAPH, 1), lambda i: (0, 0)),
        out_shape=jax.ShapeDtypeStruct((NGRAPH, 1), _f32),
        scratch_shapes=[
            pltpu.VMEM((NGRAPH, 1), _f32),
            pltpu.VMEM((NGRAPH, 1), _f32),
        ],
    )(z3, y3, dis, b3, Wl, blr, batch_p)


def kernel(x, edge_index, batch, W1, b1, W2, b2, W3, b3, Wl, bl):
    src = edge_index[0]
    dst = edge_index[1]
    pad_e = EP - NE
    src_p = jnp.concatenate([src, jnp.zeros((pad_e,), jnp.int32)])
    dst_p = jnp.concatenate(
        [dst, NP + (jnp.arange(pad_e, dtype=jnp.int32) % 128)])
    src_t = src_p.reshape(NS, CHUNKS, 1, CHUNK)
    dst_t = dst_p.reshape(NS, CHUNKS, 1, CHUNK)

    x_p = jnp.pad(x, ((0, NP - NV), (0, 0)))
    batch_p = jnp.pad(batch, (0, NP - NV),
                      constant_values=NGRAPH).reshape(GRID_N, 1, TN)

    deg_parts = _deg_call(dst_p.reshape(NC * NS, DROWS, CHUNK))
    degp = deg_parts[:, :NP, None]

    y1, dis = _t1_call(x_p, degp, W1)
    z1 = _agg_call(y1, src_t, dst_t)
    y2 = _t23_call(z1, y1, dis, b1.reshape(1, HID), W2)
    z2 = _agg_call(y2, src_t, dst_t)
    y3 = _t23_call(z2, y2, dis, b2.reshape(1, HID), W3)
    z3 = _agg_call(y3, src_t, dst_t)
    return _t4_call(z3, y3, dis, b3.reshape(1, HID), Wl,
                    bl.reshape(1, 1), batch_p)


# speedup vs baseline: 14.5955x; 1.0436x over previous
import jax
import jax.numpy as jnp
from jax import lax
from jax.experimental import pallas as pl
from jax.experimental.pallas import tpu as pltpu
from jax.experimental.pallas import tpu_sc as plsc

NC, NS, LANES = 2, 16, 16
NV = 50000
NE = 800000
FIN = 8
HID = 64
NQ = 4
HQ = HID // NQ
NPASS = NQ // NC
NGRAPH = 256

NP = 50176
NA = 51200
CHUNK = 128
CHUNKS = 392
EP = NS * CHUNKS * CHUNK
NBUF = 8
OUTER = CHUNKS // NBUF
DROWS = CHUNKS // NC

TN = 1024
GRID_N = NP // TN

_f32 = jnp.float32
_mesh = plsc.VectorSubcoreMesh(core_axis_name="c", subcore_axis_name="s",
                               num_cores=NC, num_subcores=NS)
_sc_params = pltpu.CompilerParams(needs_layout_passes=False,
                                  use_tc_tiling_on_sc=False)


def _deg_body(dst_hbm, deg_out, dslab, hist):
    ci = lax.axis_index("c")
    si = lax.axis_index("s")
    wid = si * NC + ci
    pltpu.sync_copy(dst_hbm.at[wid], dslab)
    zeros = jnp.zeros((LANES,), _f32)
    ones = jnp.ones((LANES,), _f32)

    def _zero(i, carry):
        hist[pl.ds(i * LANES, LANES)] = zeros
        return carry
    lax.fori_loop(0, NA // LANES, _zero, 0)

    def _row(r, carry):
        def _grp(q, carry2):
            idx = dslab[r, pl.ds(q * LANES, LANES)]
            plsc.addupdate_scatter(hist, [idx], ones)
            return carry2
        return lax.fori_loop(0, CHUNK // LANES, _grp, carry)
    lax.fori_loop(0, DROWS, _row, 0)

    pltpu.sync_copy(hist, deg_out.at[wid])


_deg_call = pl.kernel(
    _deg_body,
    out_type=pltpu.HBM((NC * NS, NA), _f32),
    mesh=_mesh,
    compiler_params=_sc_params,
    scratch_types=[
        pltpu.VMEM((DROWS, CHUNK), jnp.int32),
        pltpu.VMEM((NA,), _f32),
    ],
)


def _agg_body(y_hbm, src_hbm, dst_hbm, z_hbm, *scr):
    ci = lax.axis_index("c")
    si = lax.axis_index("s")
    sb = scr[0:NBUF]
    db = scr[NBUF:2 * NBUF]
    gb = scr[2 * NBUF:3 * NBUF]
    zbuf = scr[3 * NBUF]
    acc = scr[3 * NBUF + 1]
    isem = scr[3 * NBUF + 2:4 * NBUF + 2]
    gsem = scr[4 * NBUF + 2:5 * NBUF + 2]
    ssem = scr[5 * NBUF + 2:6 * NBUF + 2]

    zeros = jnp.zeros((LANES,), _f32)

    def _zb(k, carry):
        zbuf[k, pl.ds(0, LANES)] = zeros
        return carry
    lax.fori_loop(0, CHUNK, _zb, 0)

    arows = NA // NS
    zrows = NP // NS
    base = si * arows

    for p in range(NPASS):
        qi = p * NC + ci

        def _zs(t, carry):
            pltpu.sync_copy(zbuf, acc.at[pl.ds(base + t * CHUNK, CHUNK)])
            return carry
        lax.fori_loop(0, arows // CHUNK, _zs, 0)
        plsc.subcore_barrier()

        yq = y_hbm.at[qi]
        for b in range(NBUF):
            pltpu.async_copy(src_hbm.at[si, b], sb[b], isem[b])
            pltpu.async_copy(dst_hbm.at[si, b], db[b], isem[b])

        def _outer(o, carry):
            for b in range(NBUF):
                c = o * NBUF + b
                pltpu.make_async_copy(src_hbm.at[si, c], sb[b],
                                      isem[b]).wait()
                pltpu.make_async_copy(dst_hbm.at[si, c], db[b],
                                      isem[b]).wait()
                pltpu.async_copy(yq.at[sb[b].at[0]], gb[b], gsem[b])
            for b in range(NBUF):
                pltpu.make_async_copy(yq.at[sb[b].at[0]], gb[b],
                                      gsem[b]).wait()
                pltpu.async_copy(gb[b], acc.at[db[b].at[0]], ssem[b],
                                 add=True)
            for b in range(NBUF):
                c = o * NBUF + b
                pltpu.make_async_copy(gb[b], acc.at[db[b].at[0]],
                                      ssem[b]).wait()

                @pl.when(c + NBUF < CHUNKS)
                def _():
                    pltpu.async_copy(src_hbm.at[si, c + NBUF], sb[b],
                                     isem[b])
                    pltpu.async_copy(dst_hbm.at[si, c + NBUF], db[b],
                                     isem[b])
            return carry
        lax.fori_loop(0, OUTER, _outer, 0)
        plsc.subcore_barrier()

        pltpu.sync_copy(acc.at[pl.ds(si * zrows, zrows)],
                        z_hbm.at[qi, pl.ds(si * zrows, zrows)])
        plsc.subcore_barrier()


_agg_call = pl.kernel(
    _agg_body,
    out_type=pltpu.HBM((NQ, NP, HQ), _f32),
    mesh=_mesh,
    compiler_params=_sc_params,
    scratch_types=(
        [pltpu.VMEM((1, CHUNK), jnp.int32)] * (2 * NBUF)
        + [pltpu.VMEM((CHUNK, HQ), _f32)] * (NBUF + 1)
        + [pltpu.VMEM_SHARED((NA, HQ), _f32)]
        + [pltpu.SemaphoreType.DMA] * (3 * NBUF)
    ),
)


def _split_q(y):
    return [y[:, q * HQ:(q + 1) * HQ] for q in range(NQ)]


def _t1_body(x_ref, deg_ref, w_ref, y_ref, dis_ref):
    deg = jnp.sum(deg_ref[...], axis=0) + 1.0
    dis = lax.rsqrt(deg)
    y = dis * jnp.dot(x_ref[...], w_ref[...], preferred_element_type=_f32)
    for q, yq in enumerate(_split_q(y)):
        y_ref[q] = yq
    dis_ref[...] = dis


def _t1_call(x_p, degp, W1):
    return pl.pallas_call(
        _t1_body,
        grid=(GRID_N,),
        in_specs=[
            pl.BlockSpec((TN, FIN), lambda i: (i, 0)),
            pl.BlockSpec((NC * NS, TN, 1), lambda i: (0, i, 0)),
            pl.BlockSpec((FIN, HID), lambda i: (0, 0)),
        ],
        out_specs=[
            pl.BlockSpec((NQ, TN, HQ), lambda i: (0, i, 0)),
            pl.BlockSpec((TN, 1), lambda i: (i, 0)),
        ],
        out_shape=[
            jax.ShapeDtypeStruct((NQ, NP, HQ), _f32),
            jax.ShapeDtypeStruct((NP, 1), _f32),
        ],
    )(x_p, degp, W1)


def _cat_q(z_ref, y_ref):
    return jnp.concatenate([z_ref[q] + y_ref[q] for q in range(NQ)], axis=1)


def _t23_body(z_ref, y_ref, dis_ref, b_ref, w_ref, yn_ref):
    dis = dis_ref[...]
    h = jnp.maximum(dis * _cat_q(z_ref, y_ref) + b_ref[...], 0.0)
    yn = dis * jnp.dot(h, w_ref[...], preferred_element_type=_f32)
    for q, yq in enumerate(_split_q(yn)):
        yn_ref[q] = yq


def _t23_call(z, y, dis, b_prev, W_next):
    return pl.pallas_call(
        _t23_body,
        grid=(GRID_N,),
        in_specs=[
            pl.BlockSpec((NQ, TN, HQ), lambda i: (0, i, 0)),
            pl.BlockSpec((NQ, TN, HQ), lambda i: (0, i, 0)),
            pl.BlockSpec((TN, 1), lambda i: (i, 0)),
            pl.BlockSpec((1, HID), lambda i: (0, 0)),
            pl.BlockSpec((HID, HID), lambda i: (0, 0)),
        ],
        out_specs=pl.BlockSpec((NQ, TN, HQ), lambda i: (0, i, 0)),
        out_shape=jax.ShapeDtypeStruct((NQ, NP, HQ), _f32),
    )(z, y, dis, b_prev, W_next)


def _t4_body(z_ref, y_ref, dis_ref, b_ref, wl_ref, bl_ref, batch_ref,
             out_ref, sums_ref, cnt_ref):
    i = pl.program_id(0)

    @pl.when(i == 0)
    def _():
        sums_ref[...] = jnp.zeros_like(sums_ref)
        cnt_ref[...] = jnp.zeros_like(cnt_ref)

    h3 = dis_ref[...] * _cat_q(z_ref, y_ref) + b_ref[...]
    g = jnp.dot(h3, wl_ref[...], preferred_element_type=_f32)
    bb = batch_ref[0, 0, :]
    oh = (bb[:, None] ==
          lax.broadcasted_iota(jnp.int32, (TN, NGRAPH), 1)).astype(_f32)
    sums_ref[...] += lax.dot_general(oh, g, (((0,), (0,)), ((), ())),
                                     preferred_element_type=_f32)
    cnt_ref[...] += jnp.sum(oh, axis=0)[:, None]

    @pl.when(i == GRID_N - 1)
    def _():
        cnt = cnt_ref[...]
        cnt = jnp.where(cnt > 0, cnt, 1.0)
        out_ref[...] = jax.nn.sigmoid(sums_ref[...] / cnt + bl_ref[...])


def _t4_call(z3, y3, dis, b3, Wl, blr, batch_p):
    return pl.pallas_call(
        _t4_body,
        grid=(GRID_N,),
        in_specs=[
            pl.BlockSpec((NQ, TN, HQ), lambda i: (0, i, 0)),
            pl.BlockSpec((NQ, TN, HQ), lambda i: (0, i, 0)),
            pl.BlockSpec((TN, 1), lambda i: (i, 0)),
            pl.BlockSpec((1, HID), lambda i: (0, 0)),
            pl.BlockSpec((HID, 1), lambda i: (0, 0)),
            pl.BlockSpec((1, 1), lambda i: (0, 0)),
            pl.BlockSpec((1, 1, TN), lambda i: (i, 0, 0)),
        ],
        out_specs=pl.BlockSpec((NGRAPH, 1), lambda i: (0, 0)),
        out_shape=jax.ShapeDtypeStruct((NGRAPH, 1), _f32),
        scratch_shapes=[
            pltpu.VMEM((NGRAPH, 1), _f32),
            pltpu.VMEM((NGRAPH, 1), _f32),
        ],
    )(z3, y3, dis, b3, Wl, blr, batch_p)


def kernel(x, edge_index, batch, W1, b1, W2, b2, W3, b3, Wl, bl):
    src = edge_index[0]
    dst = edge_index[1]
    pad_e = EP - NE
    src_p = jnp.concatenate([src, jnp.zeros((pad_e,), jnp.int32)])
    dst_p = jnp.concatenate(
        [dst, NP + (jnp.arange(pad_e, dtype=jnp.int32) % 128)])
    src_t = src_p.reshape(NS, CHUNKS, 1, CHUNK)
    dst_t = dst_p.reshape(NS, CHUNKS, 1, CHUNK)

    x_p = jnp.pad(x, ((0, NP - NV), (0, 0)))
    batch_p = jnp.pad(batch, (0, NP - NV),
                      constant_values=NGRAPH).reshape(GRID_N, 1, TN)

    deg_parts = _deg_call(dst_p.reshape(NC * NS, DROWS, CHUNK))
    degp = deg_parts[:, :NP, None]

    y1, dis = _t1_call(x_p, degp, W1)
    z1 = _agg_call(y1, src_t, dst_t)
    y2 = _t23_call(z1, y1, dis, b1.reshape(1, HID), W2)
    z2 = _agg_call(y2, src_t, dst_t)
    y3 = _t23_call(z2, y2, dis, b2.reshape(1, HID), W3)
    z3 = _agg_call(y3, src_t, dst_t)
    return _t4_call(z3, y3, dis, b3.reshape(1, HID), Wl,
                    bl.reshape(1, 1), batch_p)


# speedup vs baseline: 20.1288x; 1.3791x over previous
import jax
import jax.numpy as jnp
from jax import lax
from jax.experimental import pallas as pl
from jax.experimental.pallas import tpu as pltpu
from jax.experimental.pallas import tpu_sc as plsc

NC, NS, LANES = 2, 16, 16
NV = 50000
NE = 800000
FIN = 8
HID = 64
NQ = 4
HQ = HID // NQ
NPASS = NQ // NC
NGRAPH = 256

NP = 50176
NA = 51200
CHUNK = 128
CHUNKS = 392
EP = NS * CHUNKS * CHUNK
NBUF = 8
OUTER = CHUNKS // NBUF
DROWS = CHUNKS // NC

TN = 1024
GRID_N = NP // TN

_f32 = jnp.float32
_mesh = plsc.VectorSubcoreMesh(core_axis_name="c", subcore_axis_name="s",
                               num_cores=NC, num_subcores=NS)
_sc_params = pltpu.CompilerParams(needs_layout_passes=False,
                                  use_tc_tiling_on_sc=False)


def _deg_body(dst_hbm, deg_out, dslab, hist):
    ci = lax.axis_index("c")
    si = lax.axis_index("s")
    wid = si * NC + ci
    pltpu.sync_copy(dst_hbm.at[wid], dslab)
    zeros = jnp.zeros((LANES,), _f32)
    ones = jnp.ones((LANES,), _f32)

    def _zero(i, carry):
        hist[pl.ds(i * LANES, LANES)] = zeros
        return carry
    lax.fori_loop(0, NA // LANES, _zero, 0)

    def _row(r, carry):
        def _grp(q, carry2):
            idx = dslab[r, pl.ds(q * LANES, LANES)]
            plsc.addupdate_scatter(hist, [idx], ones)
            return carry2
        return lax.fori_loop(0, CHUNK // LANES, _grp, carry)
    lax.fori_loop(0, DROWS, _row, 0)

    pltpu.sync_copy(hist, deg_out.at[wid])


_deg_call = pl.kernel(
    _deg_body,
    out_type=pltpu.HBM((NC * NS, NA), _f32),
    mesh=_mesh,
    compiler_params=_sc_params,
    scratch_types=[
        pltpu.VMEM((DROWS, CHUNK), jnp.int32),
        pltpu.VMEM((NA,), _f32),
    ],
)


def _agg_body(y_hbm, src_hbm, dst_hbm, z_hbm, *scr):
    ci = lax.axis_index("c")
    si = lax.axis_index("s")
    sb = scr[0:NBUF]
    db = scr[NBUF:2 * NBUF]
    gb = scr[2 * NBUF:3 * NBUF]
    zbuf = scr[3 * NBUF]
    acc = scr[3 * NBUF + 1]
    isem = scr[3 * NBUF + 2:4 * NBUF + 2]
    gsem = scr[4 * NBUF + 2:5 * NBUF + 2]
    ssem = scr[5 * NBUF + 2:6 * NBUF + 2]

    zeros = jnp.zeros((LANES,), _f32)

    def _zb(k, carry):
        zbuf[k, pl.ds(0, LANES)] = zeros
        return carry
    lax.fori_loop(0, CHUNK, _zb, 0)

    arows = NA // NS
    zrows = NP // NS
    base = si * arows

    for p in range(NPASS):
        qi = p * NC + ci

        def _zs(t, carry):
            pltpu.sync_copy(zbuf, acc.at[pl.ds(base + t * CHUNK, CHUNK)])
            return carry
        lax.fori_loop(0, arows // CHUNK, _zs, 0)
        plsc.subcore_barrier()

        yq = y_hbm.at[qi]
        for b in range(NBUF):
            pltpu.async_copy(src_hbm.at[si, b], sb[b], isem[b])
            pltpu.async_copy(dst_hbm.at[si, b], db[b], isem[b])

        def _outer(o, carry):
            for b in range(NBUF):
                c = o * NBUF + b
                pltpu.make_async_copy(src_hbm.at[si, c], sb[b],
                                      isem[b]).wait()
                pltpu.make_async_copy(dst_hbm.at[si, c], db[b],
                                      isem[b]).wait()
                pltpu.async_copy(yq.at[sb[b].at[0]], gb[b], gsem[b])
            for b in range(NBUF):
                pltpu.make_async_copy(yq.at[sb[b].at[0]], gb[b],
                                      gsem[b]).wait()
                pltpu.async_copy(gb[b], acc.at[db[b].at[0]], ssem[b],
                                 add=True)
            for b in range(NBUF):
                c = o * NBUF + b
                pltpu.make_async_copy(gb[b], acc.at[db[b].at[0]],
                                      ssem[b]).wait()

                @pl.when(c + NBUF < CHUNKS)
                def _():
                    pltpu.async_copy(src_hbm.at[si, c + NBUF], sb[b],
                                     isem[b])
                    pltpu.async_copy(dst_hbm.at[si, c + NBUF], db[b],
                                     isem[b])
            return carry
        lax.fori_loop(0, OUTER, _outer, 0)
        plsc.subcore_barrier()

        pltpu.sync_copy(acc.at[pl.ds(si * zrows, zrows)],
                        z_hbm.at[qi, pl.ds(si * zrows, zrows)])
        plsc.subcore_barrier()


_agg_call = pl.kernel(
    _agg_body,
    out_type=pltpu.HBM((NQ, NP, HQ), _f32),
    mesh=_mesh,
    compiler_params=_sc_params,
    scratch_types=(
        [pltpu.VMEM((1, CHUNK), jnp.int32)] * (2 * NBUF)
        + [pltpu.VMEM((CHUNK, HQ), _f32)] * (NBUF + 1)
        + [pltpu.VMEM_SHARED((NA, HQ), _f32)]
        + [pltpu.SemaphoreType.DMA] * (3 * NBUF)
    ),
)


def _split_q(y):
    return [y[:, q * HQ:(q + 1) * HQ] for q in range(NQ)]


def _t1_body(x_ref, deg_ref, w_ref, y_ref, dis_ref):
    deg = jnp.sum(deg_ref[...], axis=0) + 1.0
    dis = lax.rsqrt(deg)[:, None]
    y = dis * jnp.dot(x_ref[...], w_ref[...], preferred_element_type=_f32)
    for q, yq in enumerate(_split_q(y)):
        y_ref[q] = yq
    dis_ref[...] = dis


def _t1_call(x_p, degp, W1):
    return pl.pallas_call(
        _t1_body,
        grid=(GRID_N,),
        in_specs=[
            pl.BlockSpec((TN, FIN), lambda i: (i, 0)),
            pl.BlockSpec((NC * NS, TN), lambda i: (0, i)),
            pl.BlockSpec((FIN, HID), lambda i: (0, 0)),
        ],
        out_specs=[
            pl.BlockSpec((NQ, TN, HQ), lambda i: (0, i, 0)),
            pl.BlockSpec((TN, 1), lambda i: (i, 0)),
        ],
        out_shape=[
            jax.ShapeDtypeStruct((NQ, NP, HQ), _f32),
            jax.ShapeDtypeStruct((NP, 1), _f32),
        ],
    )(x_p, degp, W1)


def _cat_q(z_ref, y_ref):
    return jnp.concatenate([z_ref[q] + y_ref[q] for q in range(NQ)], axis=1)


def _t23_body(z_ref, y_ref, dis_ref, b_ref, w_ref, yn_ref):
    dis = dis_ref[...]
    h = jnp.maximum(dis * _cat_q(z_ref, y_ref) + b_ref[...], 0.0)
    yn = dis * jnp.dot(h, w_ref[...], preferred_element_type=_f32)
    for q, yq in enumerate(_split_q(yn)):
        yn_ref[q] = yq


def _t23_call(z, y, dis, b_prev, W_next):
    return pl.pallas_call(
        _t23_body,
        grid=(GRID_N,),
        in_specs=[
            pl.BlockSpec((NQ, TN, HQ), lambda i: (0, i, 0)),
            pl.BlockSpec((NQ, TN, HQ), lambda i: (0, i, 0)),
            pl.BlockSpec((TN, 1), lambda i: (i, 0)),
            pl.BlockSpec((1, HID), lambda i: (0, 0)),
            pl.BlockSpec((HID, HID), lambda i: (0, 0)),
        ],
        out_specs=pl.BlockSpec((NQ, TN, HQ), lambda i: (0, i, 0)),
        out_shape=jax.ShapeDtypeStruct((NQ, NP, HQ), _f32),
    )(z, y, dis, b_prev, W_next)


def _t4_body(z_ref, y_ref, dis_ref, b_ref, wl_ref, bl_ref, batch_ref,
             out_ref, sums_ref, cnt_ref):
    i = pl.program_id(0)

    @pl.when(i == 0)
    def _():
        sums_ref[...] = jnp.zeros_like(sums_ref)
        cnt_ref[...] = jnp.zeros_like(cnt_ref)

    h3 = dis_ref[...] * _cat_q(z_ref, y_ref) + b_ref[...]
    g = jnp.dot(h3, wl_ref[...], preferred_element_type=_f32)
    bb = batch_ref[0, 0, :]
    oh = (bb[:, None] ==
          lax.broadcasted_iota(jnp.int32, (TN, NGRAPH), 1)).astype(_f32)
    sums_ref[...] += lax.dot_general(oh, g, (((0,), (0,)), ((), ())),
                                     preferred_element_type=_f32)
    cnt_ref[...] += jnp.sum(oh, axis=0)[:, None]

    @pl.when(i == GRID_N - 1)
    def _():
        cnt = cnt_ref[...]
        cnt = jnp.where(cnt > 0, cnt, 1.0)
        out_ref[...] = jax.nn.sigmoid(sums_ref[...] / cnt + bl_ref[...])


def _t4_call(z3, y3, dis, b3, Wl, blr, batch_p):
    return pl.pallas_call(
        _t4_body,
        grid=(GRID_N,),
        in_specs=[
            pl.BlockSpec((NQ, TN, HQ), lambda i: (0, i, 0)),
            pl.BlockSpec((NQ, TN, HQ), lambda i: (0, i, 0)),
            pl.BlockSpec((TN, 1), lambda i: (i, 0)),
            pl.BlockSpec((1, HID), lambda i: (0, 0)),
            pl.BlockSpec((HID, 1), lambda i: (0, 0)),
            pl.BlockSpec((1, 1), lambda i: (0, 0)),
            pl.BlockSpec((1, 1, TN), lambda i: (i, 0, 0)),
        ],
        out_specs=pl.BlockSpec((NGRAPH, 1), lambda i: (0, 0)),
        out_shape=jax.ShapeDtypeStruct((NGRAPH, 1), _f32),
        scratch_shapes=[
            pltpu.VMEM((NGRAPH, 1), _f32),
            pltpu.VMEM((NGRAPH, 1), _f32),
        ],
    )(z3, y3, dis, b3, Wl, blr, batch_p)


def kernel(x, edge_index, batch, W1, b1, W2, b2, W3, b3, Wl, bl):
    src = edge_index[0]
    dst = edge_index[1]
    pad_e = EP - NE
    src_p = jnp.concatenate([src, jnp.zeros((pad_e,), jnp.int32)])
    dst_p = jnp.concatenate(
        [dst, NP + (jnp.arange(pad_e, dtype=jnp.int32) % 128)])
    src_t = src_p.reshape(NS, CHUNKS, 1, CHUNK)
    dst_t = dst_p.reshape(NS, CHUNKS, 1, CHUNK)

    x_p = jnp.pad(x, ((0, NP - NV), (0, 0)))
    batch_p = jnp.pad(batch, (0, NP - NV),
                      constant_values=NGRAPH).reshape(GRID_N, 1, TN)

    deg_parts = _deg_call(dst_p.reshape(NC * NS, DROWS, CHUNK))
    degp = deg_parts[:, :NP]

    y1, dis = _t1_call(x_p, degp, W1)
    z1 = _agg_call(y1, src_t, dst_t)
    y2 = _t23_call(z1, y1, dis, b1.reshape(1, HID), W2)
    z2 = _agg_call(y2, src_t, dst_t)
    y3 = _t23_call(z2, y2, dis, b2.reshape(1, HID), W3)
    z3 = _agg_call(y3, src_t, dst_t)
    return _t4_call(z3, y3, dis, b3.reshape(1, HID), Wl,
                    bl.reshape(1, 1), batch_p)


# speedup vs baseline: 20.6375x; 1.0253x over previous
import jax
import jax.numpy as jnp
from jax import lax
from jax.experimental import pallas as pl
from jax.experimental.pallas import tpu as pltpu
from jax.experimental.pallas import tpu_sc as plsc

NC, NS, LANES = 2, 16, 16
NV = 50000
NE = 800000
FIN = 8
HID = 64
NQ = 4
HQ = HID // NQ
NPASS = NQ // NC
NGRAPH = 256

NP = 50176
NA = 51200
CHUNK = 128
CHUNKS = 392
EP = NS * CHUNKS * CHUNK
NBUF = 8
OUTER = CHUNKS // NBUF
DROWS = CHUNKS // NC

TN = 1024
GRID_N = NP // TN

_f32 = jnp.float32
_mesh = plsc.VectorSubcoreMesh(core_axis_name="c", subcore_axis_name="s",
                               num_cores=NC, num_subcores=NS)
_sc_params = pltpu.CompilerParams(needs_layout_passes=False,
                                  use_tc_tiling_on_sc=False)


def _deg_body(dst_hbm, deg_out, dslab, hist):
    ci = lax.axis_index("c")
    si = lax.axis_index("s")
    wid = si * NC + ci
    pltpu.sync_copy(dst_hbm.at[wid], dslab)
    zeros = jnp.zeros((LANES,), _f32)
    ones = jnp.ones((LANES,), _f32)

    def _zero(i, carry):
        hist[pl.ds(i * LANES, LANES)] = zeros
        return carry
    lax.fori_loop(0, NA // LANES, _zero, 0)

    def _row(r, carry):
        def _grp(q, carry2):
            idx = dslab[r, pl.ds(q * LANES, LANES)]
            plsc.addupdate_scatter(hist, [idx], ones)
            return carry2
        return lax.fori_loop(0, CHUNK // LANES, _grp, carry)
    lax.fori_loop(0, DROWS, _row, 0)

    pltpu.sync_copy(hist, deg_out.at[wid])


_deg_call = pl.kernel(
    _deg_body,
    out_type=pltpu.HBM((NC * NS, NA), _f32),
    mesh=_mesh,
    compiler_params=_sc_params,
    scratch_types=[
        pltpu.VMEM((DROWS, CHUNK), jnp.int32),
        pltpu.VMEM((NA,), _f32),
    ],
)


def _agg_body(y_hbm, src_hbm, dst_hbm, z_hbm, *scr):
    ci = lax.axis_index("c")
    si = lax.axis_index("s")
    sb = scr[0:NBUF]
    db = scr[NBUF:2 * NBUF]
    gb = scr[2 * NBUF:3 * NBUF]
    zbuf = scr[3 * NBUF]
    acc = scr[3 * NBUF + 1]
    isem = scr[3 * NBUF + 2:4 * NBUF + 2]
    gsem = scr[4 * NBUF + 2:5 * NBUF + 2]
    ssem = scr[5 * NBUF + 2:6 * NBUF + 2]

    zeros = jnp.zeros((LANES,), _f32)

    def _zb(k, carry):
        zbuf[k, pl.ds(0, LANES)] = zeros
        return carry
    lax.fori_loop(0, CHUNK, _zb, 0)

    arows = NA // NS
    zrows = NP // NS
    base = si * arows

    for p in range(NPASS):
        qi = p * NC + ci

        def _zs(t, carry):
            pltpu.sync_copy(zbuf, acc.at[pl.ds(base + t * CHUNK, CHUNK)])
            return carry
        lax.fori_loop(0, arows // CHUNK, _zs, 0)
        plsc.subcore_barrier()

        yq = y_hbm.at[qi]
        for b in range(NBUF):
            pltpu.async_copy(src_hbm.at[si, b], sb[b], isem[b])
            pltpu.async_copy(dst_hbm.at[si, b], db[b], isem[b])

        def _outer(o, carry):
            for b in range(NBUF):
                c = o * NBUF + b
                pltpu.make_async_copy(src_hbm.at[si, c], sb[b],
                                      isem[b]).wait()
                pltpu.make_async_copy(dst_hbm.at[si, c], db[b],
                                      isem[b]).wait()
                pltpu.async_copy(yq.at[sb[b].at[0]], gb[b], gsem[b])
            for b in range(NBUF):
                pltpu.make_async_copy(yq.at[sb[b].at[0]], gb[b],
                                      gsem[b]).wait()
                pltpu.async_copy(gb[b], acc.at[db[b].at[0]], ssem[b],
                                 add=True)
            for b in range(NBUF):
                c = o * NBUF + b
                pltpu.make_async_copy(gb[b], acc.at[db[b].at[0]],
                                      ssem[b]).wait()

                @pl.when(c + NBUF < CHUNKS)
                def _():
                    pltpu.async_copy(src_hbm.at[si, c + NBUF], sb[b],
                                     isem[b])
                    pltpu.async_copy(dst_hbm.at[si, c + NBUF], db[b],
                                     isem[b])
            return carry
        lax.fori_loop(0, OUTER, _outer, 0)
        plsc.subcore_barrier()

        pltpu.sync_copy(acc.at[pl.ds(si * zrows, zrows)],
                        z_hbm.at[qi, pl.ds(si * zrows, zrows)])
        plsc.subcore_barrier()


_agg_call = pl.kernel(
    _agg_body,
    out_type=pltpu.HBM((NQ, NP, HQ), _f32),
    mesh=_mesh,
    compiler_params=_sc_params,
    scratch_types=(
        [pltpu.VMEM((1, CHUNK), jnp.int32)] * (2 * NBUF)
        + [pltpu.VMEM((CHUNK, HQ), _f32)] * (NBUF + 1)
        + [pltpu.VMEM_SHARED((NA, HQ), _f32)]
        + [pltpu.SemaphoreType.DMA] * (3 * NBUF)
    ),
)


TNP = TN // 8


def _t1_body(x_ref, deg_ref, w_ref, y_ref, dis_ref):
    deg = jnp.sum(deg_ref[...], axis=0) + 1.0
    dis_row = lax.rsqrt(deg)
    dis = dis_row[:, None]
    y = dis * jnp.dot(x_ref[...], w_ref[...], preferred_element_type=_f32)
    for q in range(NQ):
        y_ref[q] = y[:, q * HQ:(q + 1) * HQ]
    dis_ref[0, 0, :] = dis_row


def _t1_call(x_p, degp, W1):
    return pl.pallas_call(
        _t1_body,
        grid=(GRID_N,),
        in_specs=[
            pl.BlockSpec((TN, FIN), lambda i: (i, 0)),
            pl.BlockSpec((NC * NS, TN), lambda i: (0, i)),
            pl.BlockSpec((FIN, HID), lambda i: (0, 0)),
        ],
        out_specs=[
            pl.BlockSpec((NQ, TN, HQ), lambda i: (0, i, 0)),
            pl.BlockSpec((1, 1, TN), lambda i: (i, 0, 0)),
        ],
        out_shape=[
            jax.ShapeDtypeStruct((NQ, NP, HQ), _f32),
            jax.ShapeDtypeStruct((GRID_N, 1, TN), _f32),
        ],
    )(x_p, degp, W1)


def _cat_q(z_ref, y_ref):
    return jnp.concatenate([z_ref[q] + y_ref[q] for q in range(NQ)], axis=1)


def _t23_body(z_ref, y_ref, dis_ref, b_ref, w_ref, yn_ref):
    dis = dis_ref[0, 0, :][:, None]
    h = jnp.maximum(dis * _cat_q(z_ref, y_ref) + b_ref[...], 0.0)
    yn = dis * jnp.dot(h, w_ref[...], preferred_element_type=_f32)
    for q in range(NQ):
        yn_ref[q] = yn[:, q * HQ:(q + 1) * HQ]


def _t23_call(z, y, dis, b_prev, W_next):
    return pl.pallas_call(
        _t23_body,
        grid=(GRID_N,),
        in_specs=[
            pl.BlockSpec((NQ, TN, HQ), lambda i: (0, i, 0)),
            pl.BlockSpec((NQ, TN, HQ), lambda i: (0, i, 0)),
            pl.BlockSpec((1, 1, TN), lambda i: (i, 0, 0)),
            pl.BlockSpec((1, HID), lambda i: (0, 0)),
            pl.BlockSpec((HID, HID), lambda i: (0, 0)),
        ],
        out_specs=pl.BlockSpec((NQ, TN, HQ), lambda i: (0, i, 0)),
        out_shape=jax.ShapeDtypeStruct((NQ, NP, HQ), _f32),
    )(z, y, dis, b_prev, W_next)


def _t4_body(z_ref, y_ref, dis_ref, b_ref, wl_ref, bl_ref, batch_ref,
             out_ref, sums_ref, cnt_ref):
    i = pl.program_id(0)

    @pl.when(i == 0)
    def _():
        sums_ref[...] = jnp.zeros_like(sums_ref)
        cnt_ref[...] = jnp.zeros_like(cnt_ref)

    dis = dis_ref[0, 0, :][:, None]
    h3 = dis * _cat_q(z_ref, y_ref) + b_ref[...]
    g = jnp.dot(h3, wl_ref[...], preferred_element_type=_f32)
    bb = batch_ref[0, 0, :]
    oh = (bb[:, None] ==
          lax.broadcasted_iota(jnp.int32, (TN, NGRAPH), 1)).astype(_f32)
    sums_ref[...] += lax.dot_general(oh, g, (((0,), (0,)), ((), ())),
                                     preferred_element_type=_f32)
    cnt_ref[...] += jnp.sum(oh, axis=0)[:, None]

    @pl.when(i == GRID_N - 1)
    def _():
        cnt = cnt_ref[...]
        cnt = jnp.where(cnt > 0, cnt, 1.0)
        out_ref[...] = jax.nn.sigmoid(sums_ref[...] / cnt + bl_ref[...])


def _t4_call(z3, y3, dis, b3, Wl, blr, batch_p):
    return pl.pallas_call(
        _t4_body,
        grid=(GRID_N,),
        in_specs=[
            pl.BlockSpec((NQ, TN, HQ), lambda i: (0, i, 0)),
            pl.BlockSpec((NQ, TN, HQ), lambda i: (0, i, 0)),
            pl.BlockSpec((1, 1, TN), lambda i: (i, 0, 0)),
            pl.BlockSpec((1, HID), lambda i: (0, 0)),
            pl.BlockSpec((HID, 1), lambda i: (0, 0)),
            pl.BlockSpec((1, 1), lambda i: (0, 0)),
            pl.BlockSpec((1, 1, TN), lambda i: (i, 0, 0)),
        ],
        out_specs=pl.BlockSpec((NGRAPH, 1), lambda i: (0, 0)),
        out_shape=jax.ShapeDtypeStruct((NGRAPH, 1), _f32),
        scratch_shapes=[
            pltpu.VMEM((NGRAPH, 1), _f32),
            pltpu.VMEM((NGRAPH, 1), _f32),
        ],
    )(z3, y3, dis, b3, Wl, blr, batch_p)


def kernel(x, edge_index, batch, W1, b1, W2, b2, W3, b3, Wl, bl):
    src = edge_index[0]
    dst = edge_index[1]
    pad_e = EP - NE
    src_p = jnp.concatenate([src, jnp.zeros((pad_e,), jnp.int32)])
    dst_p = jnp.concatenate(
        [dst, NP + (jnp.arange(pad_e, dtype=jnp.int32) % 128)])
    src_t = src_p.reshape(NS, CHUNKS, 1, CHUNK)
    dst_t = dst_p.reshape(NS, CHUNKS, 1, CHUNK)

    x_p = jnp.pad(x, ((0, NP - NV), (0, 0)))
    batch_p = jnp.pad(batch, (0, NP - NV),
                      constant_values=NGRAPH).reshape(GRID_N, 1, TN)

    deg_parts = _deg_call(dst_p.reshape(NC * NS, DROWS, CHUNK))
    degp = deg_parts[:, :NP]

    y1, dis = _t1_call(x_p, degp, W1)
    z1 = _agg_call(y1, src_t, dst_t)
    y2 = _t23_call(z1, y1, dis, b1.reshape(1, HID), W2)
    z2 = _agg_call(y2, src_t, dst_t)
    y3 = _t23_call(z2, y2, dis, b2.reshape(1, HID), W3)
    z3 = _agg_call(y3, src_t, dst_t)
    return _t4_call(z3, y3, dis, b3.reshape(1, HID), Wl,
                    bl.reshape(1, 1), batch_p)
